# Initial kernel scaffold; baseline (speedup 1.0000x reference)
import jax
import jax.numpy as jnp
from jax.experimental import pallas as pl


def kernel(x, edge_index, W1, b1, W2, b2, Wc, bc):
    raise NotImplementedError("write your pallas kernel here")


# speedup vs baseline: 13.2446x; 13.2446x over previous
import functools

import jax
import jax.numpy as jnp
from jax import lax
from jax.experimental import pallas as pl
from jax.experimental.pallas import tpu as pltpu
from jax.experimental.pallas import tpu_sc as plsc

NC = 2
NS = 16

N = 100000
E = 1600000
EP = 1638400
ROWS = EP // 128
NPAD = 100352
HALF = N // 2
ACC_ROWS = 50048
DUMMY = HALF

_MESH = plsc.VectorSubcoreMesh(
    core_axis_name="c", subcore_axis_name="s", num_cores=NC, num_subcores=NS)


def _drain(src, dst, sem, n):
    for _ in range(n):
        pltpu.make_async_copy(src, dst, sem).wait()


@functools.partial(
    pl.kernel,
    out_type=jax.ShapeDtypeStruct((NC, 1, NPAD), jnp.float32),
    mesh=_MESH,
    scratch_types=[
        pltpu.VMEM((2, 8, 128), jnp.int32),
        pltpu.VMEM((128,), jnp.float32),
        pltpu.VMEM_SHARED((NPAD,), jnp.float32),
        pltpu.SemaphoreType.DMA,
        pltpu.SemaphoreType.DMA,
        pltpu.SemaphoreType.DMA,
    ],
)
def _deg_kernel(dst2d, zeros_v, degp, dstb, ones_v, deg_sh, lsem, ssem0, ssem1):
    cid = lax.axis_index("c")
    sid = lax.axis_index("s")
    for i in range(8):
        ones_v[pl.ds(i * 16, 16)] = jnp.full((16,), 1.0, jnp.float32)
    pltpu.sync_copy(zeros_v, deg_sh.at[pl.ds(sid * 6272, 6272)])
    plsc.subcore_barrier()

    nblk = 50
    tbase = cid * 6400 + sid * 400

    def idx_src(bi):
        return dst2d.at[pl.ds(tbase + bi * 8, 8)]

    def fire_load(bi, k):
        pltpu.async_copy(idx_src(bi), dstb.at[k], lsem)

    def wait_load(bi, k):
        pltpu.make_async_copy(idx_src(bi), dstb.at[k], lsem).wait()

    def scat(k, j, sem):
        return (ones_v, deg_sh.at[dstb.at[k, j]], sem)

    def process(bi, k, sem, drain_prev, prefetch=True):
        if prefetch:
            @pl.when(bi + 1 < nblk)
            def _():
                fire_load(bi + 1, (k + 1) % 2)
        if drain_prev:
            _drain(*scat(k, 0, sem), 8)
        for j in range(8):
            pltpu.async_copy(*scat(k, j, sem))

    pltpu.sync_copy(idx_src(0), dstb.at[0])
    fire_load(1, 1)
    process(0, 0, ssem0, False, prefetch=False)
    wait_load(1, 1)
    process(1, 1, ssem1, False)

    def body(i, carry):
        b0 = 2 * i
        wait_load(b0, 0)
        process(b0, 0, ssem0, True)
        wait_load(b0 + 1, 1)
        process(b0 + 1, 1, ssem1, True)
        return carry

    lax.fori_loop(1, nblk // 2, body, 0)
    _drain(*scat(0, 0, ssem0), 8)
    _drain(*scat(1, 0, ssem1), 8)
    plsc.subcore_barrier()
    pltpu.sync_copy(deg_sh.at[pl.ds(sid * 6272, 6272)],
                    degp.at[cid, 0, pl.ds(sid * 6272, 6272)])


@functools.partial(
    pl.kernel,
    out_type=jax.ShapeDtypeStruct((NC, 1, NPAD), jnp.float32),
    mesh=_MESH,
    scratch_types=[
        pltpu.VMEM((2, 8, 128), jnp.int32),
        pltpu.VMEM((2, 8, 128), jnp.int32),
        pltpu.VMEM((2, 8, 128), jnp.float32),
        pltpu.VMEM_SHARED((NPAD,), jnp.float32),
        pltpu.SemaphoreType.DMA,
        pltpu.SemaphoreType.DMA,
        pltpu.SemaphoreType.DMA,
        pltpu.SemaphoreType.DMA,
    ],
)
def _c2_kernel(src2d, dst2d, dpad, zeros_v, c2p,
               srcb, dstb, dvb, c2_sh, lsem, gsem, ssem0, ssem1):
    cid = lax.axis_index("c")
    sid = lax.axis_index("s")
    pltpu.sync_copy(zeros_v, c2_sh.at[pl.ds(sid * 6272, 6272)])
    plsc.subcore_barrier()

    nblk = 50
    tbase = cid * 6400 + sid * 400

    def loads(bi, k):
        return ((src2d.at[pl.ds(tbase + bi * 8, 8)], srcb.at[k]),
                (dst2d.at[pl.ds(tbase + bi * 8, 8)], dstb.at[k]))

    def fire_load(bi, k):
        for s, d in loads(bi, k):
            pltpu.async_copy(s, d, lsem)

    def wait_load(bi, k):
        for s, d in loads(bi, k):
            pltpu.make_async_copy(s, d, lsem).wait()

    def scat(k, j, sem):
        return (dvb.at[k, j], c2_sh.at[srcb.at[k, j]], sem)

    def process(bi, k, sem, drain_prev, prefetch=True):
        if prefetch:
            @pl.when(bi + 1 < nblk)
            def _():
                fire_load(bi + 1, (k + 1) % 2)
        if drain_prev:
            _drain(*scat(k, 0, sem), 8)
        descs = [pltpu.async_copy(dpad.at[dstb.at[k, j]], dvb.at[k, j], gsem)
                 for j in range(8)]
        for dsc in descs:
            dsc.wait()
        for j in range(8):
            pltpu.async_copy(*scat(k, j, sem))

    pltpu.sync_copy(*loads(0, 0)[0])
    pltpu.sync_copy(*loads(0, 0)[1])
    fire_load(1, 1)
    process(0, 0, ssem0, False, prefetch=False)
    wait_load(1, 1)
    process(1, 1, ssem1, False)

    def body(i, carry):
        b0 = 2 * i
        wait_load(b0, 0)
        process(b0, 0, ssem0, True)
        wait_load(b0 + 1, 1)
        process(b0 + 1, 1, ssem1, True)
        return carry

    lax.fori_loop(1, nblk // 2, body, 0)
    _drain(*scat(0, 0, ssem0), 8)
    _drain(*scat(1, 0, ssem1), 8)
    plsc.subcore_barrier()
    pltpu.sync_copy(c2_sh.at[pl.ds(sid * 6272, 6272)],
                    c2p.at[cid, 0, pl.ds(sid * 6272, 6272)])


@functools.partial(
    pl.kernel,
    out_type=jax.ShapeDtypeStruct((2, NC, ACC_ROWS, 32), jnp.float32),
    mesh=_MESH,
    compiler_params=pltpu.CompilerParams(use_tc_tiling_on_sc=False),
    scratch_types=[
        pltpu.VMEM((2, 8, 128), jnp.int32),
        pltpu.VMEM((2, 8, 128), jnp.int32),
        pltpu.VMEM((2, 8, 128), jnp.int32),
        pltpu.VMEM((4, 128, 32), jnp.float32),
        pltpu.VMEM_SHARED((ACC_ROWS, 32), jnp.float32),
        pltpu.SemaphoreType.DMA,
        [pltpu.SemaphoreType.DMA] * 4,
        [pltpu.SemaphoreType.DMA] * 4,
    ],
)
def _main_kernel(src2d, dst2d, hxd0, hxd1, zacc, acc_out,
                 srcb, dstb, dlb, rowsb, acc_sh, lsem, gsems, ssems):
    cid = lax.axis_index("c")
    sid = lax.axis_index("s")
    base = cid * HALF
    nblk = 100
    tbase = sid * 800

    def loads(bi, k):
        return ((src2d.at[pl.ds(tbase + bi * 8, 8)], srcb.at[k]),
                (dst2d.at[pl.ds(tbase + bi * 8, 8)], dstb.at[k]))

    def fire_load(bi, k):
        for s, d in loads(bi, k):
            pltpu.async_copy(s, d, lsem)

    def wait_load(bi, k):
        for s, d in loads(bi, k):
            pltpu.make_async_copy(s, d, lsem).wait()

    def gath(hxdf, k, c):
        return (hxdf.at[srcb.at[k, c]], rowsb.at[c % 4], gsems[c % 4])

    def scat(k, c):
        return (rowsb.at[c % 4], acc_sh.at[dlb.at[k, c]], ssems[c % 4])

    for f in range(2):
        hxdf = hxd0 if f == 0 else hxd1
        pltpu.sync_copy(zacc, acc_sh.at[pl.ds(sid * 3128, 3128)])
        plsc.subcore_barrier()

        def process(bi, k, first, prefetch=True):
            if prefetch:
                @pl.when(bi + 1 < nblk)
                def _():
                    fire_load(bi + 1, (k + 1) % 2)
            for c in range(3):
                if not first:
                    _drain(*scat(k, c), 1)
                pltpu.async_copy(*gath(hxdf, k, c))
            for r in range(8):
                for q in range(8):
                    v = dstb[k, r, pl.ds(q * 16, 16)]
                    m = (v >= base) & (v < base + HALF)
                    dlb[k, r, pl.ds(q * 16, 16)] = jnp.where(m, v - base, DUMMY)
            for j in range(8):
                c = j + 3
                if c < 8:
                    if not (first and c == 3):
                        _drain(*scat(k, c), 1)
                    pltpu.async_copy(*gath(hxdf, k, c))
                pltpu.make_async_copy(*gath(hxdf, k, j)[:2], gsems[j % 4]).wait()
                pltpu.async_copy(*scat(k, j))

        pltpu.sync_copy(*loads(0, 0)[0])
        pltpu.sync_copy(*loads(0, 0)[1])
        fire_load(1, 1)
        process(0, 0, True, prefetch=False)
        wait_load(1, 1)
        process(1, 1, False)

        def body(i, carry):
            b0 = 2 * i
            wait_load(b0, 0)
            process(b0, 0, False)
            wait_load(b0 + 1, 1)
            process(b0 + 1, 1, False)
            return carry

        lax.fori_loop(1, nblk // 2, body, 0)
        for c in range(4):
            _drain(*scat(1, c), 1)
        plsc.subcore_barrier()
        pltpu.sync_copy(acc_sh.at[pl.ds(sid * 3128, 3128)],
                        acc_out.at[f, cid, pl.ds(sid * 3128, 3128)])
        plsc.subcore_barrier()


def _mm_body(x_ref, w_ref, o_ref):
    o_ref[...] = jnp.dot(x_ref[...], w_ref[...],
                         preferred_element_type=jnp.float32)


def _scale_body(hx_ref, c0_ref, c1_ref, h0_ref, h1_ref, d_ref):
    d = lax.rsqrt(c0_ref[...] + c1_ref[...] + 1.0)
    hxd = hx_ref[...] * d
    h0_ref[...] = hxd[:, :32]
    h1_ref[...] = hxd[:, 32:]
    d_ref[...] = d


def _final_body(acc0, acc1, hx, d, c20, c21, b1, W2, b2, Wc, bc,
                o_ref, s_ref):
    i = pl.program_id(0)

    @pl.when(i == 0)
    def _():
        s_ref[...] = jnp.zeros_like(s_ref)

    dv = d[...]
    accw = jnp.concatenate([acc0[...], acc1[...]], axis=1)
    a1 = dv * accw + (dv * dv) * hx[...] + b1[...]
    h1 = jnp.maximum(a1, 0.0)
    c = dv * (c20[...] + c21[...] + dv)
    s_ref[...] += jnp.sum(c * h1, axis=0, keepdims=True)

    @pl.when(i == pl.num_programs(0) - 1)
    def _():
        s = s_ref[...] * (1.0 / N)
        g = jnp.dot(s, W2[...], preferred_element_type=jnp.float32) + b2[...]
        z = jnp.dot(g, Wc[...], preferred_element_type=jnp.float32) + bc[...]
        o_ref[...] = 1.0 / (1.0 + jnp.exp(-z))


def kernel(x, edge_index, W1, b1, W2, b2, Wc, bc):
    n, e = x.shape[0], edge_index.shape[1]
    src, dst = edge_index[0], edge_index[1]
    pad = EP - e
    src2d = jnp.concatenate([src, jnp.zeros((pad,), jnp.int32)]).reshape(ROWS, 128)
    dst2d = jnp.concatenate([dst, jnp.full((pad,), n, jnp.int32)]).reshape(ROWS, 128)
    zeros1 = jnp.zeros((6272,), jnp.float32)
    zacc = jnp.zeros((3128, 32), jnp.float32)

    R = 2000
    grid = (n // R,)

    hx = pl.pallas_call(
        _mm_body,
        grid=grid,
        in_specs=[pl.BlockSpec((R, 32), lambda i: (i, 0)),
                  pl.BlockSpec((32, 64), lambda i: (0, 0))],
        out_specs=pl.BlockSpec((R, 64), lambda i: (i, 0)),
        out_shape=jax.ShapeDtypeStruct((n, 64), jnp.float32),
    )(x, W1)

    degp = _deg_kernel(dst2d, zeros1)
    c0 = degp[0, 0, :n].reshape(n, 1)
    c1 = degp[1, 0, :n].reshape(n, 1)

    hxd0, hxd1, dcol = pl.pallas_call(
        _scale_body,
        grid=grid,
        in_specs=[pl.BlockSpec((R, 64), lambda i: (i, 0)),
                  pl.BlockSpec((R, 1), lambda i: (i, 0)),
                  pl.BlockSpec((R, 1), lambda i: (i, 0))],
        out_specs=[pl.BlockSpec((R, 32), lambda i: (i, 0)),
                   pl.BlockSpec((R, 32), lambda i: (i, 0)),
                   pl.BlockSpec((R, 1), lambda i: (i, 0))],
        out_shape=[jax.ShapeDtypeStruct((n, 32), jnp.float32),
                   jax.ShapeDtypeStruct((n, 32), jnp.float32),
                   jax.ShapeDtypeStruct((n, 1), jnp.float32)],
    )(hx, c0, c1)

    dpad = jnp.concatenate([dcol.reshape(n), jnp.zeros((NPAD - n,), jnp.float32)])

    accp = _main_kernel(src2d, dst2d, hxd0, hxd1, zacc)
    acc = accp[:, :, :HALF, :].reshape(2, n, 32)
    c2p = _c2_kernel(src2d, dst2d, dpad, zeros1)

    out = pl.pallas_call(
        _final_body,
        grid=grid,
        in_specs=[pl.BlockSpec((R, 32), lambda i: (i, 0)),
                  pl.BlockSpec((R, 32), lambda i: (i, 0)),
                  pl.BlockSpec((R, 64), lambda i: (i, 0)),
                  pl.BlockSpec((R, 1), lambda i: (i, 0)),
                  pl.BlockSpec((R, 1), lambda i: (i, 0)),
                  pl.BlockSpec((R, 1), lambda i: (i, 0)),
                  pl.BlockSpec((1, 64), lambda i: (0, 0)),
                  pl.BlockSpec((64, 64), lambda i: (0, 0)),
                  pl.BlockSpec((1, 64), lambda i: (0, 0)),
                  pl.BlockSpec((64, 1), lambda i: (0, 0)),
                  pl.BlockSpec((1, 1), lambda i: (0, 0))],
        out_specs=pl.BlockSpec((1, 1), lambda i: (0, 0)),
        out_shape=jax.ShapeDtypeStruct((1, 1), jnp.float32),
        scratch_shapes=[pltpu.VMEM((1, 64), jnp.float32)],
    )(acc[0], acc[1], hx, dcol,
      c2p[0, 0, :n].reshape(n, 1), c2p[1, 0, :n].reshape(n, 1),
      b1.reshape(1, 64), W2, b2.reshape(1, 64), Wc, bc.reshape(1, 1))
    return out


# speedup vs baseline: 14.8218x; 1.1191x over previous
import functools

import jax
import jax.numpy as jnp
from jax import lax
from jax.experimental import pallas as pl
from jax.experimental.pallas import tpu as pltpu
from jax.experimental.pallas import tpu_sc as plsc

NC = 2
NS = 16

N = 100000
E = 1600000
EP = 1638400
ROWS = EP // 128
NPAD = 100352
HALF = N // 2
ACC_ROWS = 50176
DUMMY = HALF

_MESH = plsc.VectorSubcoreMesh(
    core_axis_name="c", subcore_axis_name="s", num_cores=NC, num_subcores=NS)


def _drain(src, dst, sem, n):
    for _ in range(n):
        pltpu.make_async_copy(src, dst, sem).wait()


@functools.partial(
    pl.kernel,
    out_type=jax.ShapeDtypeStruct((NC, 1, NPAD), jnp.float32),
    mesh=_MESH,
    scratch_types=[
        pltpu.VMEM((2, 8, 128), jnp.int32),
        pltpu.VMEM((128,), jnp.float32),
        pltpu.VMEM_SHARED((NPAD,), jnp.float32),
        pltpu.SemaphoreType.DMA,
        pltpu.SemaphoreType.DMA,
        pltpu.SemaphoreType.DMA,
    ],
)
def _deg_kernel(dst2d, zeros_v, degp, dstb, ones_v, deg_sh, lsem, ssem0, ssem1):
    cid = lax.axis_index("c")
    sid = lax.axis_index("s")
    for i in range(8):
        ones_v[pl.ds(i * 16, 16)] = jnp.full((16,), 1.0, jnp.float32)
    pltpu.sync_copy(zeros_v, deg_sh.at[pl.ds(sid * 6272, 6272)])
    plsc.subcore_barrier()

    nblk = 50
    tbase = cid * 6400 + sid * 400

    def idx_src(bi):
        return dst2d.at[pl.ds(tbase + bi * 8, 8)]

    def fire_load(bi, k):
        pltpu.async_copy(idx_src(bi), dstb.at[k], lsem)

    def wait_load(bi, k):
        pltpu.make_async_copy(idx_src(bi), dstb.at[k], lsem).wait()

    def scat(k, j, sem):
        return (ones_v, deg_sh.at[dstb.at[k, j]], sem)

    def process(bi, k, sem, drain_prev, prefetch=True):
        if prefetch:
            @pl.when(bi + 1 < nblk)
            def _():
                fire_load(bi + 1, (k + 1) % 2)
        if drain_prev:
            _drain(*scat(k, 0, sem), 8)
        for j in range(8):
            pltpu.async_copy(*scat(k, j, sem))

    pltpu.sync_copy(idx_src(0), dstb.at[0])
    fire_load(1, 1)
    process(0, 0, ssem0, False, prefetch=False)
    wait_load(1, 1)
    process(1, 1, ssem1, False)

    def body(i, carry):
        b0 = 2 * i
        wait_load(b0, 0)
        process(b0, 0, ssem0, True)
        wait_load(b0 + 1, 1)
        process(b0 + 1, 1, ssem1, True)
        return carry

    lax.fori_loop(1, nblk // 2, body, 0)
    _drain(*scat(0, 0, ssem0), 8)
    _drain(*scat(1, 0, ssem1), 8)
    plsc.subcore_barrier()
    pltpu.sync_copy(deg_sh.at[pl.ds(sid * 6272, 6272)],
                    degp.at[cid, 0, pl.ds(sid * 6272, 6272)])


@functools.partial(
    pl.kernel,
    out_type=jax.ShapeDtypeStruct((NC, 1, NPAD), jnp.float32),
    mesh=_MESH,
    scratch_types=[
        pltpu.VMEM((2, 8, 128), jnp.int32),
        pltpu.VMEM((2, 8, 128), jnp.int32),
        pltpu.VMEM((2, 8, 128), jnp.float32),
        pltpu.VMEM_SHARED((NPAD,), jnp.float32),
        pltpu.SemaphoreType.DMA,
        pltpu.SemaphoreType.DMA,
        pltpu.SemaphoreType.DMA,
        pltpu.SemaphoreType.DMA,
    ],
)
def _c2_kernel(src2d, dst2d, dpad, zeros_v, c2p,
               srcb, dstb, dvb, c2_sh, lsem, gsem, ssem0, ssem1):
    cid = lax.axis_index("c")
    sid = lax.axis_index("s")
    pltpu.sync_copy(zeros_v, c2_sh.at[pl.ds(sid * 6272, 6272)])
    plsc.subcore_barrier()

    nblk = 50
    tbase = cid * 6400 + sid * 400

    def loads(bi, k):
        return ((src2d.at[pl.ds(tbase + bi * 8, 8)], srcb.at[k]),
                (dst2d.at[pl.ds(tbase + bi * 8, 8)], dstb.at[k]))

    def fire_load(bi, k):
        for s, d in loads(bi, k):
            pltpu.async_copy(s, d, lsem)

    def wait_load(bi, k):
        for s, d in loads(bi, k):
            pltpu.make_async_copy(s, d, lsem).wait()

    def scat(k, j, sem):
        return (dvb.at[k, j], c2_sh.at[srcb.at[k, j]], sem)

    def process(bi, k, sem, drain_prev, prefetch=True):
        if prefetch:
            @pl.when(bi + 1 < nblk)
            def _():
                fire_load(bi + 1, (k + 1) % 2)
        if drain_prev:
            _drain(*scat(k, 0, sem), 8)
        descs = [pltpu.async_copy(dpad.at[dstb.at[k, j]], dvb.at[k, j], gsem)
                 for j in range(8)]
        for dsc in descs:
            dsc.wait()
        for j in range(8):
            pltpu.async_copy(*scat(k, j, sem))

    pltpu.sync_copy(*loads(0, 0)[0])
    pltpu.sync_copy(*loads(0, 0)[1])
    fire_load(1, 1)
    process(0, 0, ssem0, False, prefetch=False)
    wait_load(1, 1)
    process(1, 1, ssem1, False)

    def body(i, carry):
        b0 = 2 * i
        wait_load(b0, 0)
        process(b0, 0, ssem0, True)
        wait_load(b0 + 1, 1)
        process(b0 + 1, 1, ssem1, True)
        return carry

    lax.fori_loop(1, nblk // 2, body, 0)
    _drain(*scat(0, 0, ssem0), 8)
    _drain(*scat(1, 0, ssem1), 8)
    plsc.subcore_barrier()
    pltpu.sync_copy(c2_sh.at[pl.ds(sid * 6272, 6272)],
                    c2p.at[cid, 0, pl.ds(sid * 6272, 6272)])


@functools.partial(
    pl.kernel,
    out_type=jax.ShapeDtypeStruct((2, NC, ACC_ROWS, 32), jnp.float32),
    mesh=_MESH,
    compiler_params=pltpu.CompilerParams(use_tc_tiling_on_sc=False),
    scratch_types=[
        pltpu.VMEM((2, 8, 128), jnp.int32),
        pltpu.VMEM((2, 8, 128), jnp.int32),
        pltpu.VMEM((2, 8, 128), jnp.int32),
        pltpu.VMEM((4, 128, 32), jnp.float32),
        pltpu.VMEM_SHARED((ACC_ROWS, 32), jnp.float32),
        pltpu.SemaphoreType.DMA,
        [pltpu.SemaphoreType.DMA] * 4,
        [pltpu.SemaphoreType.DMA] * 4,
    ],
)
def _main_kernel(src2d, dst2d, hxd0, hxd1, zacc, acc_out,
                 srcb, dstb, dlb, rowsb, acc_sh, lsem, gsems, ssems):
    cid = lax.axis_index("c")
    sid = lax.axis_index("s")
    base = cid * HALF
    nblk = 100
    tbase = sid * 800

    def loads(bi, k):
        return ((src2d.at[pl.ds(tbase + bi * 8, 8)], srcb.at[k]),
                (dst2d.at[pl.ds(tbase + bi * 8, 8)], dstb.at[k]))

    def fire_load(bi, k):
        for s, d in loads(bi, k):
            pltpu.async_copy(s, d, lsem)

    def wait_load(bi, k):
        for s, d in loads(bi, k):
            pltpu.make_async_copy(s, d, lsem).wait()

    def gath(hxdf, k, c):
        return (hxdf.at[srcb.at[k, c]], rowsb.at[c % 4], gsems[c % 4])

    def scat(k, c):
        return (rowsb.at[c % 4], acc_sh.at[dlb.at[k, c]], ssems[c % 4])

    for f in range(2):
        hxdf = hxd0 if f == 0 else hxd1
        pltpu.sync_copy(zacc, acc_sh.at[pl.ds(sid * 3136, 3136)])
        plsc.subcore_barrier()

        def process(bi, k, first, prefetch=True):
            if prefetch:
                @pl.when(bi + 1 < nblk)
                def _():
                    fire_load(bi + 1, (k + 1) % 2)
            for c in range(3):
                if not first:
                    _drain(*scat(k, c), 1)
                pltpu.async_copy(*gath(hxdf, k, c))
            for r in range(8):
                for q in range(8):
                    v = dstb[k, r, pl.ds(q * 16, 16)]
                    m = (v >= base) & (v < base + HALF)
                    dlb[k, r, pl.ds(q * 16, 16)] = jnp.where(
                        m, v - base, DUMMY + (v & 127))
            for j in range(8):
                c = j + 3
                if c < 8:
                    if not (first and c == 3):
                        _drain(*scat(k, c), 1)
                    pltpu.async_copy(*gath(hxdf, k, c))
                pltpu.make_async_copy(*gath(hxdf, k, j)[:2], gsems[j % 4]).wait()
                pltpu.async_copy(*scat(k, j))

        pltpu.sync_copy(*loads(0, 0)[0])
        pltpu.sync_copy(*loads(0, 0)[1])
        fire_load(1, 1)
        process(0, 0, True, prefetch=False)
        wait_load(1, 1)
        process(1, 1, False)

        def body(i, carry):
            b0 = 2 * i
            wait_load(b0, 0)
            process(b0, 0, False)
            wait_load(b0 + 1, 1)
            process(b0 + 1, 1, False)
            return carry

        lax.fori_loop(1, nblk // 2, body, 0)
        for c in range(4):
            _drain(*scat(1, c), 1)
        plsc.subcore_barrier()
        pltpu.sync_copy(acc_sh.at[pl.ds(sid * 3136, 3136)],
                        acc_out.at[f, cid, pl.ds(sid * 3136, 3136)])
        plsc.subcore_barrier()


def _mm_body(x_ref, w_ref, o_ref):
    o_ref[...] = jnp.dot(x_ref[...], w_ref[...],
                         preferred_element_type=jnp.float32)


def _scale_body(hx_ref, c0_ref, c1_ref, h0_ref, h1_ref, d_ref):
    d = lax.rsqrt(c0_ref[...] + c1_ref[...] + 1.0)
    hxd = hx_ref[...] * d
    h0_ref[...] = hxd[:, :32]
    h1_ref[...] = hxd[:, 32:]
    d_ref[...] = d


def _final_body(acc0, acc1, hx, d, c20, c21, b1, W2, b2, Wc, bc,
                o_ref, s_ref):
    i = pl.program_id(0)

    @pl.when(i == 0)
    def _():
        s_ref[...] = jnp.zeros_like(s_ref)

    dv = d[...]
    accw = jnp.concatenate([acc0[...], acc1[...]], axis=1)
    a1 = dv * accw + (dv * dv) * hx[...] + b1[...]
    h1 = jnp.maximum(a1, 0.0)
    c = dv * (c20[...] + c21[...] + dv)
    s_ref[...] += jnp.sum(c * h1, axis=0, keepdims=True)

    @pl.when(i == pl.num_programs(0) - 1)
    def _():
        s = s_ref[...] * (1.0 / N)
        g = jnp.dot(s, W2[...], preferred_element_type=jnp.float32) + b2[...]
        z = jnp.dot(g, Wc[...], preferred_element_type=jnp.float32) + bc[...]
        o_ref[...] = 1.0 / (1.0 + jnp.exp(-z))


def kernel(x, edge_index, W1, b1, W2, b2, Wc, bc):
    n, e = x.shape[0], edge_index.shape[1]
    src, dst = edge_index[0], edge_index[1]
    pad = EP - e
    src2d = jnp.concatenate([src, jnp.zeros((pad,), jnp.int32)]).reshape(ROWS, 128)
    dst2d = jnp.concatenate([dst, jnp.full((pad,), n, jnp.int32)]).reshape(ROWS, 128)
    zeros1 = jnp.zeros((6272,), jnp.float32)
    zacc = jnp.zeros((3136, 32), jnp.float32)

    R = 2000
    grid = (n // R,)

    hx = pl.pallas_call(
        _mm_body,
        grid=grid,
        in_specs=[pl.BlockSpec((R, 32), lambda i: (i, 0)),
                  pl.BlockSpec((32, 64), lambda i: (0, 0))],
        out_specs=pl.BlockSpec((R, 64), lambda i: (i, 0)),
        out_shape=jax.ShapeDtypeStruct((n, 64), jnp.float32),
    )(x, W1)

    degp = _deg_kernel(dst2d, zeros1)
    c0 = degp[0, 0, :n].reshape(n, 1)
    c1 = degp[1, 0, :n].reshape(n, 1)

    hxd0, hxd1, dcol = pl.pallas_call(
        _scale_body,
        grid=grid,
        in_specs=[pl.BlockSpec((R, 64), lambda i: (i, 0)),
                  pl.BlockSpec((R, 1), lambda i: (i, 0)),
                  pl.BlockSpec((R, 1), lambda i: (i, 0))],
        out_specs=[pl.BlockSpec((R, 32), lambda i: (i, 0)),
                   pl.BlockSpec((R, 32), lambda i: (i, 0)),
                   pl.BlockSpec((R, 1), lambda i: (i, 0))],
        out_shape=[jax.ShapeDtypeStruct((n, 32), jnp.float32),
                   jax.ShapeDtypeStruct((n, 32), jnp.float32),
                   jax.ShapeDtypeStruct((n, 1), jnp.float32)],
    )(hx, c0, c1)

    dpad = jnp.concatenate([dcol.reshape(n), jnp.zeros((NPAD - n,), jnp.float32)])

    accp = _main_kernel(src2d, dst2d, hxd0, hxd1, zacc)
    acc = accp[:, :, :HALF, :].reshape(2, n, 32)
    c2p = _c2_kernel(src2d, dst2d, dpad, zeros1)

    out = pl.pallas_call(
        _final_body,
        grid=grid,
        in_specs=[pl.BlockSpec((R, 32), lambda i: (i, 0)),
                  pl.BlockSpec((R, 32), lambda i: (i, 0)),
                  pl.BlockSpec((R, 64), lambda i: (i, 0)),
                  pl.BlockSpec((R, 1), lambda i: (i, 0)),
                  pl.BlockSpec((R, 1), lambda i: (i, 0)),
                  pl.BlockSpec((R, 1), lambda i: (i, 0)),
                  pl.BlockSpec((1, 64), lambda i: (0, 0)),
                  pl.BlockSpec((64, 64), lambda i: (0, 0)),
                  pl.BlockSpec((1, 64), lambda i: (0, 0)),
                  pl.BlockSpec((64, 1), lambda i: (0, 0)),
                  pl.BlockSpec((1, 1), lambda i: (0, 0))],
        out_specs=pl.BlockSpec((1, 1), lambda i: (0, 0)),
        out_shape=jax.ShapeDtypeStruct((1, 1), jnp.float32),
        scratch_shapes=[pltpu.VMEM((1, 64), jnp.float32)],
    )(acc[0], acc[1], hx, dcol,
      c2p[0, 0, :n].reshape(n, 1), c2p[1, 0, :n].reshape(n, 1),
      b1.reshape(1, 64), W2, b2.reshape(1, 64), Wc, bc.reshape(1, 1))
    return out


# speedup vs baseline: 15.8162x; 1.0671x over previous
import functools

import jax
import jax.numpy as jnp
from jax import lax
from jax.experimental import pallas as pl
from jax.experimental.pallas import tpu as pltpu
from jax.experimental.pallas import tpu_sc as plsc

NC = 2
NS = 16

N = 100000
E = 1600000
EP = 1638400
ROWS = EP // 128
NPAD = 100352
HALF = N // 2
ACC_ROWS = 50176
DUMMY = HALF

_MESH = plsc.VectorSubcoreMesh(
    core_axis_name="c", subcore_axis_name="s", num_cores=NC, num_subcores=NS)


def _drain(src, dst, sem, n):
    for _ in range(n):
        pltpu.make_async_copy(src, dst, sem).wait()


@functools.partial(
    pl.kernel,
    out_type=jax.ShapeDtypeStruct((NC, 1, NPAD), jnp.float32),
    mesh=_MESH,
    scratch_types=[
        pltpu.VMEM((2, 8, 128), jnp.int32),
        pltpu.VMEM((128,), jnp.float32),
        pltpu.VMEM_SHARED((NPAD,), jnp.float32),
        pltpu.SemaphoreType.DMA,
        pltpu.SemaphoreType.DMA,
        pltpu.SemaphoreType.DMA,
    ],
)
def _deg_kernel(dst2d, zeros_v, degp, dstb, ones_v, deg_sh, lsem, ssem0, ssem1):
    cid = lax.axis_index("c")
    sid = lax.axis_index("s")
    for i in range(8):
        ones_v[pl.ds(i * 16, 16)] = jnp.full((16,), 1.0, jnp.float32)
    pltpu.sync_copy(zeros_v, deg_sh.at[pl.ds(sid * 6272, 6272)])
    plsc.subcore_barrier()

    nblk = 50
    tbase = cid * 6400 + sid * 400

    def idx_src(bi):
        return dst2d.at[pl.ds(tbase + bi * 8, 8)]

    def fire_load(bi, k):
        pltpu.async_copy(idx_src(bi), dstb.at[k], lsem)

    def wait_load(bi, k):
        pltpu.make_async_copy(idx_src(bi), dstb.at[k], lsem).wait()

    def scat(k, j, sem):
        return (ones_v, deg_sh.at[dstb.at[k, j]], sem)

    def process(bi, k, sem, drain_prev, prefetch=True):
        if prefetch:
            @pl.when(bi + 1 < nblk)
            def _():
                fire_load(bi + 1, (k + 1) % 2)
        if drain_prev:
            _drain(*scat(k, 0, sem), 8)
        for j in range(8):
            pltpu.async_copy(*scat(k, j, sem), add=True)

    pltpu.sync_copy(idx_src(0), dstb.at[0])
    fire_load(1, 1)
    process(0, 0, ssem0, False, prefetch=False)
    wait_load(1, 1)
    process(1, 1, ssem1, False)

    def body(i, carry):
        b0 = 2 * i
        wait_load(b0, 0)
        process(b0, 0, ssem0, True)
        wait_load(b0 + 1, 1)
        process(b0 + 1, 1, ssem1, True)
        return carry

    lax.fori_loop(1, nblk // 2, body, 0)
    _drain(*scat(0, 0, ssem0), 8)
    _drain(*scat(1, 0, ssem1), 8)
    plsc.subcore_barrier()
    pltpu.sync_copy(deg_sh.at[pl.ds(sid * 6272, 6272)],
                    degp.at[cid, 0, pl.ds(sid * 6272, 6272)])


@functools.partial(
    pl.kernel,
    out_type=jax.ShapeDtypeStruct((NC, 1, NPAD), jnp.float32),
    mesh=_MESH,
    scratch_types=[
        pltpu.VMEM((2, 8, 128), jnp.int32),
        pltpu.VMEM((2, 8, 128), jnp.int32),
        pltpu.VMEM((2, 8, 128), jnp.float32),
        pltpu.VMEM_SHARED((NPAD,), jnp.float32),
        pltpu.SemaphoreType.DMA,
        pltpu.SemaphoreType.DMA,
        pltpu.SemaphoreType.DMA,
        pltpu.SemaphoreType.DMA,
    ],
)
def _c2_kernel(src2d, dst2d, dpad, zeros_v, c2p,
               srcb, dstb, dvb, c2_sh, lsem, gsem, ssem0, ssem1):
    cid = lax.axis_index("c")
    sid = lax.axis_index("s")
    pltpu.sync_copy(zeros_v, c2_sh.at[pl.ds(sid * 6272, 6272)])
    plsc.subcore_barrier()

    nblk = 50
    tbase = cid * 6400 + sid * 400

    def loads(bi, k):
        return ((src2d.at[pl.ds(tbase + bi * 8, 8)], srcb.at[k]),
                (dst2d.at[pl.ds(tbase + bi * 8, 8)], dstb.at[k]))

    def fire_load(bi, k):
        for s, d in loads(bi, k):
            pltpu.async_copy(s, d, lsem)

    def wait_load(bi, k):
        for s, d in loads(bi, k):
            pltpu.make_async_copy(s, d, lsem).wait()

    def scat(k, j, sem):
        return (dvb.at[k, j], c2_sh.at[srcb.at[k, j]], sem)

    def process(bi, k, sem, drain_prev, prefetch=True):
        if prefetch:
            @pl.when(bi + 1 < nblk)
            def _():
                fire_load(bi + 1, (k + 1) % 2)
        if drain_prev:
            _drain(*scat(k, 0, sem), 8)
        descs = [pltpu.async_copy(dpad.at[dstb.at[k, j]], dvb.at[k, j], gsem)
                 for j in range(8)]
        for dsc in descs:
            dsc.wait()
        for j in range(8):
            pltpu.async_copy(*scat(k, j, sem), add=True)

    pltpu.sync_copy(*loads(0, 0)[0])
    pltpu.sync_copy(*loads(0, 0)[1])
    fire_load(1, 1)
    process(0, 0, ssem0, False, prefetch=False)
    wait_load(1, 1)
    process(1, 1, ssem1, False)

    def body(i, carry):
        b0 = 2 * i
        wait_load(b0, 0)
        process(b0, 0, ssem0, True)
        wait_load(b0 + 1, 1)
        process(b0 + 1, 1, ssem1, True)
        return carry

    lax.fori_loop(1, nblk // 2, body, 0)
    _drain(*scat(0, 0, ssem0), 8)
    _drain(*scat(1, 0, ssem1), 8)
    plsc.subcore_barrier()
    pltpu.sync_copy(c2_sh.at[pl.ds(sid * 6272, 6272)],
                    c2p.at[cid, 0, pl.ds(sid * 6272, 6272)])


@functools.partial(
    pl.kernel,
    out_type=jax.ShapeDtypeStruct((4, NC, NPAD, 16), jnp.float32),
    mesh=_MESH,
    compiler_params=pltpu.CompilerParams(use_tc_tiling_on_sc=False),
    scratch_types=[
        pltpu.VMEM((2, 8, 128), jnp.int32),
        pltpu.VMEM((2, 8, 128), jnp.int32),
        pltpu.VMEM((4, 128, 16), jnp.float32),
        pltpu.VMEM_SHARED((NPAD, 16), jnp.float32),
        pltpu.SemaphoreType.DMA,
        [pltpu.SemaphoreType.DMA] * 4,
        [pltpu.SemaphoreType.DMA] * 4,
    ],
)
def _main_kernel(src2d, dst2d, hxq0, hxq1, hxq2, hxq3, zacc, acc_out,
                 srcb, dstb, rowsb, acc_sh, lsem, gsems, ssems):
    cid = lax.axis_index("c")
    sid = lax.axis_index("s")
    nblk = 50
    tbase = cid * 6400 + sid * 400

    def loads(bi, k):
        return ((src2d.at[pl.ds(tbase + bi * 8, 8)], srcb.at[k]),
                (dst2d.at[pl.ds(tbase + bi * 8, 8)], dstb.at[k]))

    def fire_load(bi, k):
        for s, d in loads(bi, k):
            pltpu.async_copy(s, d, lsem)

    def wait_load(bi, k):
        for s, d in loads(bi, k):
            pltpu.make_async_copy(s, d, lsem).wait()

    def gath(hxdf, k, c):
        return (hxdf.at[srcb.at[k, c]], rowsb.at[c % 4], gsems[c % 4])

    def scat(k, c):
        return (rowsb.at[c % 4], acc_sh.at[dstb.at[k, c]], ssems[c % 4])

    for f in range(4):
        hxdf = (hxq0, hxq1, hxq2, hxq3)[f]
        pltpu.sync_copy(zacc, acc_sh.at[pl.ds(sid * 6272, 6272)])
        plsc.subcore_barrier()

        def process(bi, k, first, prefetch=True):
            for c in range(3):
                if not first:
                    _drain(*scat(k, c), 1)
                pltpu.async_copy(*gath(hxdf, k, c))
            for j in range(8):
                c = j + 3
                if c < 8:
                    if not (first and c == 3):
                        _drain(*scat(k, c), 1)
                    pltpu.async_copy(*gath(hxdf, k, c))
                if j == 1 and prefetch:
                    @pl.when(bi + 1 < nblk)
                    def _():
                        fire_load(bi + 1, (k + 1) % 2)
                pltpu.make_async_copy(*gath(hxdf, k, j)[:2], gsems[j % 4]).wait()
                pltpu.async_copy(*scat(k, j), add=True)

        pltpu.sync_copy(*loads(0, 0)[0])
        pltpu.sync_copy(*loads(0, 0)[1])
        fire_load(1, 1)
        process(0, 0, True, prefetch=False)
        wait_load(1, 1)
        process(1, 1, False)

        def body(i, carry):
            b0 = 2 * i
            wait_load(b0, 0)
            process(b0, 0, False)
            wait_load(b0 + 1, 1)
            process(b0 + 1, 1, False)
            return carry

        lax.fori_loop(1, nblk // 2, body, 0)
        for c in range(4):
            _drain(*scat(1, c + 4), 1)
        plsc.subcore_barrier()
        pltpu.sync_copy(acc_sh.at[pl.ds(sid * 6272, 6272)],
                        acc_out.at[f, cid, pl.ds(sid * 6272, 6272)])
        plsc.subcore_barrier()


def _mm_body(x_ref, w_ref, o_ref):
    o_ref[...] = jnp.dot(x_ref[...], w_ref[...],
                         preferred_element_type=jnp.float32)


def _scale_body(hx_ref, c0_ref, c1_ref, h0_ref, h1_ref, h2_ref, h3_ref,
                d_ref):
    d = lax.rsqrt(c0_ref[...] + c1_ref[...] + 1.0)
    hxd = hx_ref[...] * d
    h0_ref[...] = hxd[:, 0:16]
    h1_ref[...] = hxd[:, 16:32]
    h2_ref[...] = hxd[:, 32:48]
    h3_ref[...] = hxd[:, 48:64]
    d_ref[...] = d


def _final_body(a0a, a0b, a1a, a1b, a2a, a2b, a3a, a3b, hx, d, c20, c21,
                b1, W2, b2, Wc, bc, o_ref, s_ref):
    i = pl.program_id(0)

    @pl.when(i == 0)
    def _():
        s_ref[...] = jnp.zeros_like(s_ref)

    dv = d[...]
    accw = jnp.concatenate([a0a[...] + a0b[...], a1a[...] + a1b[...],
                            a2a[...] + a2b[...], a3a[...] + a3b[...]], axis=1)
    a1 = dv * accw + (dv * dv) * hx[...] + b1[...]
    h1 = jnp.maximum(a1, 0.0)
    c = dv * (c20[...] + c21[...] + dv)
    s_ref[...] += jnp.sum(c * h1, axis=0, keepdims=True)

    @pl.when(i == pl.num_programs(0) - 1)
    def _():
        s = s_ref[...] * (1.0 / N)
        g = jnp.dot(s, W2[...], preferred_element_type=jnp.float32) + b2[...]
        z = jnp.dot(g, Wc[...], preferred_element_type=jnp.float32) + bc[...]
        o_ref[...] = 1.0 / (1.0 + jnp.exp(-z))


def kernel(x, edge_index, W1, b1, W2, b2, Wc, bc):
    n, e = x.shape[0], edge_index.shape[1]
    src, dst = edge_index[0], edge_index[1]
    pad = EP - e
    src2d = jnp.concatenate([src, jnp.zeros((pad,), jnp.int32)]).reshape(ROWS, 128)
    dpadv = n + (jnp.arange(pad, dtype=jnp.int32) & 127)
    dst2d = jnp.concatenate([dst, dpadv]).reshape(ROWS, 128)
    zeros1 = jnp.zeros((6272,), jnp.float32)
    zacc = jnp.zeros((6272, 16), jnp.float32)

    R = 2000
    grid = (n // R,)

    hx = pl.pallas_call(
        _mm_body,
        grid=grid,
        in_specs=[pl.BlockSpec((R, 32), lambda i: (i, 0)),
                  pl.BlockSpec((32, 64), lambda i: (0, 0))],
        out_specs=pl.BlockSpec((R, 64), lambda i: (i, 0)),
        out_shape=jax.ShapeDtypeStruct((n, 64), jnp.float32),
    )(x, W1)

    degp = _deg_kernel(dst2d, zeros1)
    c0 = degp[0, 0, :n].reshape(n, 1)
    c1 = degp[1, 0, :n].reshape(n, 1)

    hxq0, hxq1, hxq2, hxq3, dcol = pl.pallas_call(
        _scale_body,
        grid=grid,
        in_specs=[pl.BlockSpec((R, 64), lambda i: (i, 0)),
                  pl.BlockSpec((R, 1), lambda i: (i, 0)),
                  pl.BlockSpec((R, 1), lambda i: (i, 0))],
        out_specs=[pl.BlockSpec((R, 16), lambda i: (i, 0))] * 4
        + [pl.BlockSpec((R, 1), lambda i: (i, 0))],
        out_shape=[jax.ShapeDtypeStruct((n, 16), jnp.float32)] * 4
        + [jax.ShapeDtypeStruct((n, 1), jnp.float32)],
    )(hx, c0, c1)

    dpad = jnp.concatenate([dcol.reshape(n), jnp.zeros((NPAD - n,), jnp.float32)])

    accp = _main_kernel(src2d, dst2d, hxq0, hxq1, hxq2, hxq3, zacc)
    c2p = _c2_kernel(src2d, dst2d, dpad, zeros1)

    out = pl.pallas_call(
        _final_body,
        grid=grid,
        in_specs=[pl.BlockSpec((R, 16), lambda i: (i, 0))] * 8
                 + [pl.BlockSpec((R, 64), lambda i: (i, 0)),
                  pl.BlockSpec((R, 1), lambda i: (i, 0)),
                  pl.BlockSpec((R, 1), lambda i: (i, 0)),
                  pl.BlockSpec((R, 1), lambda i: (i, 0)),
                  pl.BlockSpec((1, 64), lambda i: (0, 0)),
                  pl.BlockSpec((64, 64), lambda i: (0, 0)),
                  pl.BlockSpec((1, 64), lambda i: (0, 0)),
                  pl.BlockSpec((64, 1), lambda i: (0, 0)),
                  pl.BlockSpec((1, 1), lambda i: (0, 0))],
        out_specs=pl.BlockSpec((1, 1), lambda i: (0, 0)),
        out_shape=jax.ShapeDtypeStruct((1, 1), jnp.float32),
        scratch_shapes=[pltpu.VMEM((1, 64), jnp.float32)],
    )(accp[0, 0, :n], accp[0, 1, :n], accp[1, 0, :n], accp[1, 1, :n],
      accp[2, 0, :n], accp[2, 1, :n], accp[3, 0, :n], accp[3, 1, :n],
      hx, dcol,
      c2p[0, 0, :n].reshape(n, 1), c2p[1, 0, :n].reshape(n, 1),
      b1.reshape(1, 64), W2, b2.reshape(1, 64), Wc, bc.reshape(1, 1))
    return out


# speedup vs baseline: 15.8379x; 1.0014x over previous
import functools

import jax
import jax.numpy as jnp
from jax import lax
from jax.experimental import pallas as pl
from jax.experimental.pallas import tpu as pltpu
from jax.experimental.pallas import tpu_sc as plsc

NC = 2
NS = 16

N = 100000
E = 1600000
EP = 1638400
ROWS = EP // 128
NPAD = 100352
HALF = N // 2
ACC_ROWS = 50176
DUMMY = HALF

_MESH = plsc.VectorSubcoreMesh(
    core_axis_name="c", subcore_axis_name="s", num_cores=NC, num_subcores=NS)


def _drain(src, dst, sem, n):
    for _ in range(n):
        pltpu.make_async_copy(src, dst, sem).wait()


@functools.partial(
    pl.kernel,
    out_type=jax.ShapeDtypeStruct((NC, 1, NPAD), jnp.float32),
    mesh=_MESH,
    scratch_types=[
        pltpu.VMEM((2, 8, 128), jnp.int32),
        pltpu.VMEM((128,), jnp.float32),
        pltpu.VMEM_SHARED((NPAD,), jnp.float32),
        pltpu.SemaphoreType.DMA,
        pltpu.SemaphoreType.DMA,
        pltpu.SemaphoreType.DMA,
    ],
)
def _deg_kernel(dst2d, zeros_v, degp, dstb, ones_v, deg_sh, lsem, ssem0, ssem1):
    cid = lax.axis_index("c")
    sid = lax.axis_index("s")
    for i in range(8):
        ones_v[pl.ds(i * 16, 16)] = jnp.full((16,), 1.0, jnp.float32)
    pltpu.sync_copy(zeros_v, deg_sh.at[pl.ds(sid * 6272, 6272)])
    plsc.subcore_barrier()

    nblk = 50
    tbase = cid * 6400 + sid * 400

    def idx_src(bi):
        return dst2d.at[pl.ds(tbase + bi * 8, 8)]

    def fire_load(bi, k):
        pltpu.async_copy(idx_src(bi), dstb.at[k], lsem)

    def wait_load(bi, k):
        pltpu.make_async_copy(idx_src(bi), dstb.at[k], lsem).wait()

    def scat(k, j, sem):
        return (ones_v, deg_sh.at[dstb.at[k, j]], sem)

    def process(bi, k, sem, drain_prev, prefetch=True):
        if prefetch:
            @pl.when(bi + 1 < nblk)
            def _():
                fire_load(bi + 1, (k + 1) % 2)
        if drain_prev:
            _drain(*scat(k, 0, sem), 8)
        for j in range(8):
            pltpu.async_copy(*scat(k, j, sem), add=True)

    pltpu.sync_copy(idx_src(0), dstb.at[0])
    fire_load(1, 1)
    process(0, 0, ssem0, False, prefetch=False)
    wait_load(1, 1)
    process(1, 1, ssem1, False)

    def body(i, carry):
        b0 = 2 * i
        wait_load(b0, 0)
        process(b0, 0, ssem0, True)
        wait_load(b0 + 1, 1)
        process(b0 + 1, 1, ssem1, True)
        return carry

    lax.fori_loop(1, nblk // 2, body, 0)
    _drain(*scat(0, 0, ssem0), 8)
    _drain(*scat(1, 0, ssem1), 8)
    plsc.subcore_barrier()
    pltpu.sync_copy(deg_sh.at[pl.ds(sid * 6272, 6272)],
                    degp.at[cid, 0, pl.ds(sid * 6272, 6272)])


@functools.partial(
    pl.kernel,
    out_type=jax.ShapeDtypeStruct((NC, 1, NPAD), jnp.float32),
    mesh=_MESH,
    scratch_types=[
        pltpu.VMEM((2, 8, 128), jnp.int32),
        pltpu.VMEM((2, 8, 128), jnp.int32),
        pltpu.VMEM((2, 8, 128), jnp.float32),
        pltpu.VMEM_SHARED((NPAD,), jnp.float32),
        pltpu.SemaphoreType.DMA,
        pltpu.SemaphoreType.DMA,
        pltpu.SemaphoreType.DMA,
        pltpu.SemaphoreType.DMA,
    ],
)
def _c2_kernel(src2d, dst2d, dpad, zeros_v, c2p,
               srcb, dstb, dvb, c2_sh, lsem, gsem, ssem0, ssem1):
    cid = lax.axis_index("c")
    sid = lax.axis_index("s")
    pltpu.sync_copy(zeros_v, c2_sh.at[pl.ds(sid * 6272, 6272)])
    plsc.subcore_barrier()

    nblk = 50
    tbase = cid * 6400 + sid * 400

    def loads(bi, k):
        return ((src2d.at[pl.ds(tbase + bi * 8, 8)], srcb.at[k]),
                (dst2d.at[pl.ds(tbase + bi * 8, 8)], dstb.at[k]))

    def fire_load(bi, k):
        for s, d in loads(bi, k):
            pltpu.async_copy(s, d, lsem)

    def wait_load(bi, k):
        for s, d in loads(bi, k):
            pltpu.make_async_copy(s, d, lsem).wait()

    def scat(k, j, sem):
        return (dvb.at[k, j], c2_sh.at[srcb.at[k, j]], sem)

    def process(bi, k, sem, drain_prev, prefetch=True):
        if prefetch:
            @pl.when(bi + 1 < nblk)
            def _():
                fire_load(bi + 1, (k + 1) % 2)
        if drain_prev:
            _drain(*scat(k, 0, sem), 8)
        descs = [pltpu.async_copy(dpad.at[dstb.at[k, j]], dvb.at[k, j], gsem)
                 for j in range(8)]
        for dsc in descs:
            dsc.wait()
        for j in range(8):
            pltpu.async_copy(*scat(k, j, sem), add=True)

    pltpu.sync_copy(*loads(0, 0)[0])
    pltpu.sync_copy(*loads(0, 0)[1])
    fire_load(1, 1)
    process(0, 0, ssem0, False, prefetch=False)
    wait_load(1, 1)
    process(1, 1, ssem1, False)

    def body(i, carry):
        b0 = 2 * i
        wait_load(b0, 0)
        process(b0, 0, ssem0, True)
        wait_load(b0 + 1, 1)
        process(b0 + 1, 1, ssem1, True)
        return carry

    lax.fori_loop(1, nblk // 2, body, 0)
    _drain(*scat(0, 0, ssem0), 8)
    _drain(*scat(1, 0, ssem1), 8)
    plsc.subcore_barrier()
    pltpu.sync_copy(c2_sh.at[pl.ds(sid * 6272, 6272)],
                    c2p.at[cid, 0, pl.ds(sid * 6272, 6272)])


@functools.partial(
    pl.kernel,
    out_type=jax.ShapeDtypeStruct((4, NC, NPAD, 16), jnp.float32),
    mesh=_MESH,
    compiler_params=pltpu.CompilerParams(use_tc_tiling_on_sc=False),
    scratch_types=[
        pltpu.VMEM((2, 8, 128), jnp.int32),
        pltpu.VMEM((2, 8, 128), jnp.int32),
        pltpu.VMEM((8, 128, 16), jnp.float32),
        pltpu.VMEM_SHARED((NPAD, 16), jnp.float32),
        pltpu.SemaphoreType.DMA,
        [pltpu.SemaphoreType.DMA] * 8,
        [pltpu.SemaphoreType.DMA] * 8,
    ],
)
def _main_kernel(src2d, dst2d, hxq0, hxq1, hxq2, hxq3, zacc, acc_out,
                 srcb, dstb, rowsb, acc_sh, lsem, gsems, ssems):
    cid = lax.axis_index("c")
    sid = lax.axis_index("s")
    nblk = 50
    tbase = cid * 6400 + sid * 400

    def loads(bi, k):
        return ((src2d.at[pl.ds(tbase + bi * 8, 8)], srcb.at[k]),
                (dst2d.at[pl.ds(tbase + bi * 8, 8)], dstb.at[k]))

    def fire_load(bi, k):
        for s, d in loads(bi, k):
            pltpu.async_copy(s, d, lsem)

    def wait_load(bi, k):
        for s, d in loads(bi, k):
            pltpu.make_async_copy(s, d, lsem).wait()

    def gath(hxdf, k, c):
        return (hxdf.at[srcb.at[k, c]], rowsb.at[c % 8], gsems[c % 8])

    def scat(k, c):
        return (rowsb.at[c % 8], acc_sh.at[dstb.at[k, c]], ssems[c % 8])

    for f in range(4):
        hxdf = (hxq0, hxq1, hxq2, hxq3)[f]
        pltpu.sync_copy(zacc, acc_sh.at[pl.ds(sid * 6272, 6272)])
        plsc.subcore_barrier()

        def process(bi, k, first, prefetch=True):
            for c in range(8):
                if not first:
                    _drain(*scat(k, c), 1)
                pltpu.async_copy(*gath(hxdf, k, c))
            if prefetch:
                @pl.when(bi + 1 < nblk)
                def _():
                    fire_load(bi + 1, (k + 1) % 2)
            for j in range(8):
                pltpu.make_async_copy(*gath(hxdf, k, j)[:2], gsems[j % 8]).wait()
                pltpu.async_copy(*scat(k, j), add=True)

        pltpu.sync_copy(*loads(0, 0)[0])
        pltpu.sync_copy(*loads(0, 0)[1])
        fire_load(1, 1)
        process(0, 0, True, prefetch=False)
        wait_load(1, 1)
        process(1, 1, False)

        def body(i, carry):
            b0 = 2 * i
            wait_load(b0, 0)
            process(b0, 0, False)
            wait_load(b0 + 1, 1)
            process(b0 + 1, 1, False)
            return carry

        lax.fori_loop(1, nblk // 2, body, 0)
        for c in range(8):
            _drain(*scat(1, c), 1)
        plsc.subcore_barrier()
        pltpu.sync_copy(acc_sh.at[pl.ds(sid * 6272, 6272)],
                        acc_out.at[f, cid, pl.ds(sid * 6272, 6272)])
        plsc.subcore_barrier()


def _mm_body(x_ref, w_ref, o_ref):
    o_ref[...] = jnp.dot(x_ref[...], w_ref[...],
                         preferred_element_type=jnp.float32)


def _scale_body(hx_ref, c0_ref, c1_ref, h0_ref, h1_ref, h2_ref, h3_ref,
                d_ref):
    d = lax.rsqrt(c0_ref[...] + c1_ref[...] + 1.0)
    hxd = hx_ref[...] * d
    h0_ref[...] = hxd[:, 0:16]
    h1_ref[...] = hxd[:, 16:32]
    h2_ref[...] = hxd[:, 32:48]
    h3_ref[...] = hxd[:, 48:64]
    d_ref[...] = d


def _final_body(a0a, a0b, a1a, a1b, a2a, a2b, a3a, a3b, hx, d, c20, c21,
                b1, W2, b2, Wc, bc, o_ref, s_ref):
    i = pl.program_id(0)

    @pl.when(i == 0)
    def _():
        s_ref[...] = jnp.zeros_like(s_ref)

    dv = d[...]
    accw = jnp.concatenate([a0a[...] + a0b[...], a1a[...] + a1b[...],
                            a2a[...] + a2b[...], a3a[...] + a3b[...]], axis=1)
    a1 = dv * accw + (dv * dv) * hx[...] + b1[...]
    h1 = jnp.maximum(a1, 0.0)
    c = dv * (c20[...] + c21[...] + dv)
    s_ref[...] += jnp.sum(c * h1, axis=0, keepdims=True)

    @pl.when(i == pl.num_programs(0) - 1)
    def _():
        s = s_ref[...] * (1.0 / N)
        g = jnp.dot(s, W2[...], preferred_element_type=jnp.float32) + b2[...]
        z = jnp.dot(g, Wc[...], preferred_element_type=jnp.float32) + bc[...]
        o_ref[...] = 1.0 / (1.0 + jnp.exp(-z))


def kernel(x, edge_index, W1, b1, W2, b2, Wc, bc):
    n, e = x.shape[0], edge_index.shape[1]
    src, dst = edge_index[0], edge_index[1]
    pad = EP - e
    src2d = jnp.concatenate([src, jnp.zeros((pad,), jnp.int32)]).reshape(ROWS, 128)
    dpadv = n + (jnp.arange(pad, dtype=jnp.int32) & 127)
    dst2d = jnp.concatenate([dst, dpadv]).reshape(ROWS, 128)
    zeros1 = jnp.zeros((6272,), jnp.float32)
    zacc = jnp.zeros((6272, 16), jnp.float32)

    R = 2000
    grid = (n // R,)

    hx = pl.pallas_call(
        _mm_body,
        grid=grid,
        in_specs=[pl.BlockSpec((R, 32), lambda i: (i, 0)),
                  pl.BlockSpec((32, 64), lambda i: (0, 0))],
        out_specs=pl.BlockSpec((R, 64), lambda i: (i, 0)),
        out_shape=jax.ShapeDtypeStruct((n, 64), jnp.float32),
    )(x, W1)

    degp = _deg_kernel(dst2d, zeros1)
    c0 = degp[0, 0, :n].reshape(n, 1)
    c1 = degp[1, 0, :n].reshape(n, 1)

    hxq0, hxq1, hxq2, hxq3, dcol = pl.pallas_call(
        _scale_body,
        grid=grid,
        in_specs=[pl.BlockSpec((R, 64), lambda i: (i, 0)),
                  pl.BlockSpec((R, 1), lambda i: (i, 0)),
                  pl.BlockSpec((R, 1), lambda i: (i, 0))],
        out_specs=[pl.BlockSpec((R, 16), lambda i: (i, 0))] * 4
        + [pl.BlockSpec((R, 1), lambda i: (i, 0))],
        out_shape=[jax.ShapeDtypeStruct((n, 16), jnp.float32)] * 4
        + [jax.ShapeDtypeStruct((n, 1), jnp.float32)],
    )(hx, c0, c1)

    dpad = jnp.concatenate([dcol.reshape(n), jnp.zeros((NPAD - n,), jnp.float32)])

    accp = _main_kernel(src2d, dst2d, hxq0, hxq1, hxq2, hxq3, zacc)
    c2p = _c2_kernel(src2d, dst2d, dpad, zeros1)

    out = pl.pallas_call(
        _final_body,
        grid=grid,
        in_specs=[pl.BlockSpec((R, 16), lambda i: (i, 0))] * 8
                 + [pl.BlockSpec((R, 64), lambda i: (i, 0)),
                  pl.BlockSpec((R, 1), lambda i: (i, 0)),
                  pl.BlockSpec((R, 1), lambda i: (i, 0)),
                  pl.BlockSpec((R, 1), lambda i: (i, 0)),
                  pl.BlockSpec((1, 64), lambda i: (0, 0)),
                  pl.BlockSpec((64, 64), lambda i: (0, 0)),
                  pl.BlockSpec((1, 64), lambda i: (0, 0)),
                  pl.BlockSpec((64, 1), lambda i: (0, 0)),
                  pl.BlockSpec((1, 1), lambda i: (0, 0))],
        out_specs=pl.BlockSpec((1, 1), lambda i: (0, 0)),
        out_shape=jax.ShapeDtypeStruct((1, 1), jnp.float32),
        scratch_shapes=[pltpu.VMEM((1, 64), jnp.float32)],
    )(accp[0, 0, :n], accp[0, 1, :n], accp[1, 0, :n], accp[1, 1, :n],
      accp[2, 0, :n], accp[2, 1, :n], accp[3, 0, :n], accp[3, 1, :n],
      hx, dcol,
      c2p[0, 0, :n].reshape(n, 1), c2p[1, 0, :n].reshape(n, 1),
      b1.reshape(1, 64), W2, b2.reshape(1, 64), Wc, bc.reshape(1, 1))
    return out
